# Initial kernel scaffold; baseline (speedup 1.0000x reference)
import jax
import jax.numpy as jnp
from jax.experimental import pallas as pl


def kernel(x, conv_w, conv_b, embed):
    raise NotImplementedError("write your pallas kernel here")


# speedup vs baseline: 1.4846x; 1.4846x over previous
import functools

import jax
import jax.numpy as jnp
from jax import lax
from jax.experimental import pallas as pl
from jax.experimental.pallas import tpu as pltpu
from jax.experimental.pallas import tpu_sc as plsc

DIM, K = 32, 8192
NB = 1024
KB = 2048


def _vq_dist_body(xT_ref, w_ref, b_ref, eT_ref, ind_ref, mind_ref):
    xpT = jnp.dot(w_ref[...].astype(jnp.bfloat16),
                  xT_ref[...].astype(jnp.bfloat16),
                  preferred_element_type=jnp.float32) + b_ref[...]
    f2 = jnp.sum(xpT * xpT, axis=0, keepdims=True)
    run_min = None
    run_idx = None
    for j in range(K // KB):
        eT = eT_ref[pl.ds(j * KB, KB), :]
        e2 = jnp.sum(eT * eT, axis=1, keepdims=True)
        sT = jnp.dot(eT.astype(jnp.bfloat16), xpT.astype(jnp.bfloat16),
                     preferred_element_type=jnp.float32)
        d = f2 - 2.0 * sT + e2
        bmin = jnp.min(d, axis=0, keepdims=True)
        iota = lax.broadcasted_iota(jnp.int32, (KB, NB), 0) + j * KB
        cand = jnp.where(d == bmin, iota, K)
        bidx = jnp.min(cand, axis=0, keepdims=True)
        if j == 0:
            run_min, run_idx = bmin, bidx
        else:
            better = bmin < run_min
            run_min = jnp.where(better, bmin, run_min)
            run_idx = jnp.where(better, bidx, run_idx)
    ind_ref[...] = run_idx.reshape(1, 1, NB)
    mind_ref[...] = run_min.reshape(1, 1, NB)


def _vq_distance_argmin(xT, conv_w, conv_b, embed_T):
    n = xT.shape[1]
    nt = n // NB
    return pl.pallas_call(
        _vq_dist_body,
        grid=(nt,),
        in_specs=[
            pl.BlockSpec((xT.shape[0], NB), lambda i: (0, i)),
            pl.BlockSpec(conv_w.shape, lambda i: (0, 0)),
            pl.BlockSpec((DIM, 1), lambda i: (0, 0)),
            pl.BlockSpec((K, DIM), lambda i: (0, 0)),
        ],
        out_specs=[
            pl.BlockSpec((1, 1, NB), lambda i: (i, 0, 0)),
            pl.BlockSpec((1, 1, NB), lambda i: (i, 0, 0)),
        ],
        out_shape=[
            jax.ShapeDtypeStruct((nt, 1, NB), jnp.int32),
            jax.ShapeDtypeStruct((nt, 1, NB), jnp.float32),
        ],
    )(xT, conv_w, conv_b, embed_T)


def _make_sc_gather(n):
    info = plsc.get_sparse_core_info()
    nc, ns = info.num_cores, info.num_subcores
    nw = nc * ns
    b_per_w = n // nw
    mesh = plsc.VectorSubcoreMesh(core_axis_name="c", subcore_axis_name="s")

    @functools.partial(
        pl.kernel,
        mesh=mesh,
        compiler_params=pltpu.CompilerParams(use_tc_tiling_on_sc=False),
        out_type=jax.ShapeDtypeStruct((n, DIM), jnp.float32),
        scratch_types=[
            pltpu.VMEM((b_per_w,), jnp.int32),
            pltpu.VMEM((b_per_w, DIM), jnp.float32),
            pltpu.SemaphoreType.DMA,
        ],
    )
    def gather_kernel(table_hbm, idx_hbm, out_hbm, idx_v, rows_v, sem):
        wid = lax.axis_index("s") * nc + lax.axis_index("c")
        base = wid * b_per_w
        pltpu.sync_copy(idx_hbm.at[pl.ds(base, b_per_w)], idx_v)
        pltpu.async_copy(table_hbm.at[idx_v], rows_v, sem).wait()
        pltpu.sync_copy(rows_v, out_hbm.at[pl.ds(base, b_per_w)])

    return gather_kernel


def kernel(x, conv_w, conv_b, embed):
    b, c, h, w = x.shape
    n = b * h * w
    xT = x.astype(jnp.float32).transpose(1, 0, 2, 3).reshape(c, n)
    embed_T = embed.T
    ind3, mind3 = _vq_distance_argmin(
        xT, conv_w, conv_b.reshape(DIM, 1), embed_T)
    ind = ind3.reshape(n)
    quantize = _make_sc_gather(n)(embed_T, ind)
    diff = jnp.sum(mind3) / (n * DIM)
    quantize_t = quantize.reshape(b, h, w, DIM).transpose(0, 3, 1, 2)
    return (quantize_t, diff, ind.reshape(b, h, w))
